# baseline (speedup 1.0000x reference)
import functools

import jax
import jax.numpy as jnp
from jax.experimental import pallas as pl
from jax.experimental.pallas import tpu as pltpu

_LANE = 128
_PART = 1024


def _se_kernel(x_ref, w1_ref, b1_ref, w2_ref, b2_ref, out_ref,
               stash_ref, pool_ref, gate_ref, *, nb, nt, ct, c_tot, hw,
               inv_hw):
    b = pl.program_id(1)
    t = pl.program_id(2)
    par = jax.lax.rem(b, 2)
    row = t * ct

    @pl.when(b > 0)
    def _():
        xt = stash_ref[pl.ds(row, ct), :]
        out_ref[0] = xt * gate_ref[pl.ds((1 - par) * c_tot + row, ct), 0:1]

    @pl.when(b < nb)
    def _():
        x = x_ref[0]
        stash_ref[pl.ds(row, ct), :] = x
        part = x[:, 0:_PART]
        for j in range(1, hw // _PART):
            part = part + x[:, j * _PART:(j + 1) * _PART]
        psum = jnp.sum(part, axis=-1, keepdims=True) * inv_hw
        pool_ref[pl.ds(par * c_tot + row, ct), :] = jnp.broadcast_to(
            psum, (ct, _LANE))

        @pl.when(t == nt - 1)
        def _():
            p = pool_ref[pl.ds(par * c_tot, c_tot), 0:1]
            h = jnp.dot(w1_ref[...], p, preferred_element_type=jnp.float32)
            h = jnp.maximum(h + b1_ref[...], 0.0)
            g = jnp.dot(w2_ref[...], h, preferred_element_type=jnp.float32)
            g = jax.nn.sigmoid(g + b2_ref[...])
            gate_ref[pl.ds(par * c_tot, c_tot), :] = jnp.broadcast_to(
                g, (c_tot, _LANE))


def kernel(x_nchw, w1, b1, w2, b2):
    B, C, H, W = x_nchw.shape
    HW = H * W
    Cr = w1.shape[0]

    x_flat = x_nchw.reshape(B, C, HW)
    b1c = b1.reshape(Cr, 1)
    b2c = b2.reshape(C, 1)

    ct = max(8, min(C, (4 * 1024 * 1024) // (HW * 4) // 8 * 8))
    while C % ct != 0:
        ct //= 2
    nt = C // ct

    cores = 2 if B % 2 == 0 else 1
    nb = B // cores

    out_flat = pl.pallas_call(
        functools.partial(_se_kernel, nb=nb, nt=nt, ct=ct, c_tot=C, hw=HW,
                          inv_hw=1.0 / HW),
        out_shape=jax.ShapeDtypeStruct((B, C, HW), x_nchw.dtype),
        grid_spec=pltpu.PrefetchScalarGridSpec(
            num_scalar_prefetch=0,
            grid=(cores, nb + 1, nt),
            in_specs=[
                pl.BlockSpec(
                    (1, ct, HW),
                    lambda c, b, t: (c * nb + jnp.minimum(b, nb - 1),
                                     jnp.where(b == nb, nt - 1, t), 0)),
                pl.BlockSpec((Cr, C), lambda c, b, t: (0, 0)),
                pl.BlockSpec((Cr, 1), lambda c, b, t: (0, 0)),
                pl.BlockSpec((C, Cr), lambda c, b, t: (0, 0)),
                pl.BlockSpec((C, 1), lambda c, b, t: (0, 0)),
            ],
            out_specs=pl.BlockSpec(
                (1, ct, HW),
                lambda c, b, t: (c * nb + jnp.maximum(b - 1, 0),
                                 jnp.where(b == 0, 0, t), 0)),
            scratch_shapes=[
                pltpu.VMEM((C, HW), jnp.float32),
                pltpu.VMEM((2 * C, _LANE), jnp.float32),
                pltpu.VMEM((2 * C, _LANE), jnp.float32),
            ],
        ),
        compiler_params=pltpu.CompilerParams(
            dimension_semantics=("parallel", "arbitrary", "arbitrary"),
            vmem_limit_bytes=60 * 1024 * 1024),
        cost_estimate=pl.CostEstimate(
            flops=3 * B * C * HW + 4 * B * C * Cr,
            transcendentals=B * C,
            bytes_accessed=2 * B * C * HW * 4 + 2 * C * Cr * 4),
    )(x_flat, w1, b1c, w2, b2c)

    return out_flat.reshape(B, C, H, W)


# speedup vs baseline: 1.0067x; 1.0067x over previous
import functools

import jax
import jax.numpy as jnp
from jax.experimental import pallas as pl
from jax.experimental.pallas import tpu as pltpu

_LANE = 128
_PART = 1024


def _se_kernel(x_ref, w1_ref, b1_ref, w2_ref, b2_ref, out_ref,
               stash_ref, pool_ref, gate_ref, *, nb, nt, ct, c_tot, hw,
               inv_hw):
    b = pl.program_id(1)
    t = pl.program_id(2)
    par = jax.lax.rem(b, 2)
    row = t * ct

    @pl.when(b > 0)
    def _():
        xt = stash_ref[pl.ds(row, ct), :]
        out_ref[0] = xt * gate_ref[pl.ds((1 - par) * c_tot + row, ct), 0:1]

    @pl.when(b < nb)
    def _():
        x = x_ref[0]
        stash_ref[pl.ds(row, ct), :] = x
        part = x[:, 0:_PART]
        for j in range(1, hw // _PART):
            part = part + x[:, j * _PART:(j + 1) * _PART]
        psum = jnp.sum(part, axis=-1, keepdims=True) * inv_hw
        pool_ref[pl.ds(par * c_tot + row, ct), :] = jnp.broadcast_to(
            psum, (ct, _LANE))

        @pl.when(t == nt - 1)
        def _():
            p = pool_ref[pl.ds(par * c_tot, c_tot), 0:1]
            h = jnp.dot(w1_ref[...], p, preferred_element_type=jnp.float32)
            h = jnp.maximum(h + b1_ref[...], 0.0)
            g = jnp.dot(w2_ref[...], h, preferred_element_type=jnp.float32)
            g = jax.nn.sigmoid(g + b2_ref[...])
            gate_ref[pl.ds(par * c_tot, c_tot), :] = jnp.broadcast_to(
                g, (c_tot, _LANE))


def kernel(x_nchw, w1, b1, w2, b2):
    B, C, H, W = x_nchw.shape
    HW = H * W
    Cr = w1.shape[0]

    x_flat = x_nchw.reshape(B, C, HW)
    b1c = b1.reshape(Cr, 1)
    b2c = b2.reshape(C, 1)

    ct = max(8, min(C, (4 * 1024 * 1024) // (HW * 4) // 8 * 8))
    while C % ct != 0:
        ct //= 2
    nt = C // ct

    cores = 1
    nb = B // cores

    out_flat = pl.pallas_call(
        functools.partial(_se_kernel, nb=nb, nt=nt, ct=ct, c_tot=C, hw=HW,
                          inv_hw=1.0 / HW),
        out_shape=jax.ShapeDtypeStruct((B, C, HW), x_nchw.dtype),
        grid_spec=pltpu.PrefetchScalarGridSpec(
            num_scalar_prefetch=0,
            grid=(cores, nb + 1, nt),
            in_specs=[
                pl.BlockSpec(
                    (1, ct, HW),
                    lambda c, b, t: (c * nb + jnp.minimum(b, nb - 1),
                                     jnp.where(b == nb, nt - 1, t), 0)),
                pl.BlockSpec((Cr, C), lambda c, b, t: (0, 0)),
                pl.BlockSpec((Cr, 1), lambda c, b, t: (0, 0)),
                pl.BlockSpec((C, Cr), lambda c, b, t: (0, 0)),
                pl.BlockSpec((C, 1), lambda c, b, t: (0, 0)),
            ],
            out_specs=pl.BlockSpec(
                (1, ct, HW),
                lambda c, b, t: (c * nb + jnp.maximum(b - 1, 0),
                                 jnp.where(b == 0, 0, t), 0)),
            scratch_shapes=[
                pltpu.VMEM((C, HW), jnp.float32),
                pltpu.VMEM((2 * C, _LANE), jnp.float32),
                pltpu.VMEM((2 * C, _LANE), jnp.float32),
            ],
        ),
        compiler_params=pltpu.CompilerParams(
            dimension_semantics=("parallel", "arbitrary", "arbitrary"),
            vmem_limit_bytes=60 * 1024 * 1024),
        cost_estimate=pl.CostEstimate(
            flops=3 * B * C * HW + 4 * B * C * Cr,
            transcendentals=B * C,
            bytes_accessed=2 * B * C * HW * 4 + 2 * C * Cr * 4),
    )(x_flat, w1, b1c, w2, b2c)

    return out_flat.reshape(B, C, H, W)


# speedup vs baseline: 1.0151x; 1.0083x over previous
import functools

import jax
import jax.numpy as jnp
from jax.experimental import pallas as pl
from jax.experimental.pallas import tpu as pltpu

_LANE = 128
_PART = 1024


def _se_kernel(x_ref, w1_ref, b1_ref, w2_ref, b2_ref, out_ref,
               stash_ref, pool_ref, gate_ref, *, nb, nt, ct, c_tot, hw,
               inv_hw):
    b = pl.program_id(1)
    t = pl.program_id(2)
    par = jax.lax.rem(b, 2)
    row = t * ct

    @pl.when(b > 0)
    def _():
        xt = stash_ref[pl.ds(row, ct), :]
        out_ref[0] = xt * gate_ref[pl.ds((1 - par) * c_tot + row, ct), 0:1]

    @pl.when(b < nb)
    def _():
        x = x_ref[0]
        stash_ref[pl.ds(row, ct), :] = x
        part = x[:, 0:_PART]
        for j in range(1, hw // _PART):
            part = part + x[:, j * _PART:(j + 1) * _PART]
        psum = jnp.sum(part, axis=-1, keepdims=True) * inv_hw
        pool_ref[pl.ds(par * c_tot + row, ct), :] = jnp.broadcast_to(
            psum, (ct, _LANE))

        @pl.when(t == nt - 1)
        def _():
            p = pool_ref[pl.ds(par * c_tot, c_tot), 0:1]
            h = jnp.dot(w1_ref[...], p, preferred_element_type=jnp.float32)
            h = jnp.maximum(h + b1_ref[...], 0.0)
            g = jnp.dot(w2_ref[...], h, preferred_element_type=jnp.float32)
            g = jax.nn.sigmoid(g + b2_ref[...])
            gate_ref[pl.ds(par * c_tot, c_tot), :] = jnp.broadcast_to(
                g, (c_tot, _LANE))


def kernel(x_nchw, w1, b1, w2, b2):
    B, C, H, W = x_nchw.shape
    HW = H * W
    Cr = w1.shape[0]

    x_flat = x_nchw.reshape(B, C, HW)
    b1c = b1.reshape(Cr, 1)
    b2c = b2.reshape(C, 1)

    ct = max(8, min(C, (8 * 1024 * 1024) // (HW * 4) // 8 * 8))
    while C % ct != 0:
        ct //= 2
    nt = C // ct

    cores = 1
    nb = B // cores

    out_flat = pl.pallas_call(
        functools.partial(_se_kernel, nb=nb, nt=nt, ct=ct, c_tot=C, hw=HW,
                          inv_hw=1.0 / HW),
        out_shape=jax.ShapeDtypeStruct((B, C, HW), x_nchw.dtype),
        grid_spec=pltpu.PrefetchScalarGridSpec(
            num_scalar_prefetch=0,
            grid=(cores, nb + 1, nt),
            in_specs=[
                pl.BlockSpec(
                    (1, ct, HW),
                    lambda c, b, t: (c * nb + jnp.minimum(b, nb - 1),
                                     jnp.where(b == nb, nt - 1, t), 0)),
                pl.BlockSpec((Cr, C), lambda c, b, t: (0, 0)),
                pl.BlockSpec((Cr, 1), lambda c, b, t: (0, 0)),
                pl.BlockSpec((C, Cr), lambda c, b, t: (0, 0)),
                pl.BlockSpec((C, 1), lambda c, b, t: (0, 0)),
            ],
            out_specs=pl.BlockSpec(
                (1, ct, HW),
                lambda c, b, t: (c * nb + jnp.maximum(b - 1, 0),
                                 jnp.where(b == 0, 0, t), 0)),
            scratch_shapes=[
                pltpu.VMEM((C, HW), jnp.float32),
                pltpu.VMEM((2 * C, _LANE), jnp.float32),
                pltpu.VMEM((2 * C, _LANE), jnp.float32),
            ],
        ),
        compiler_params=pltpu.CompilerParams(
            dimension_semantics=("parallel", "arbitrary", "arbitrary"),
            vmem_limit_bytes=60 * 1024 * 1024),
        cost_estimate=pl.CostEstimate(
            flops=3 * B * C * HW + 4 * B * C * Cr,
            transcendentals=B * C,
            bytes_accessed=2 * B * C * HW * 4 + 2 * C * Cr * 4),
    )(x_flat, w1, b1c, w2, b2c)

    return out_flat.reshape(B, C, H, W)
